# baseline (speedup 1.0000x reference)
import functools

import jax
import jax.numpy as jnp
from jax import lax
from jax.experimental import pallas as pl
from jax.experimental.pallas import tpu as pltpu
from jax.experimental.pallas import tpu_sc as plsc

N = 10000
E = 320000
D = 128

NC = 2
NS = 16
NW = NC * NS
C = 128
CPW = 80
NCHUNKP = NW * CPW
EPAD = NCHUNKP * C - E
NACC = N + 8
NCNT = 10240
RPT = 624
HCH = CPW // 2

_f32 = jnp.float32


def _zeros16():
    return jnp.zeros((16,), _f32)


def _make_sc_segsum(with_count):
    out_type = [jax.ShapeDtypeStruct((NC, N, D), _f32)]
    if with_count:
        out_type.append(jax.ShapeDtypeStruct((NCNT,), _f32))
        out_type.append(jax.ShapeDtypeStruct((NCNT,), _f32))

    scratch_types = [
        pltpu.VMEM_SHARED((NACC, D), _f32),
        pltpu.VMEM_SHARED((NCNT,), _f32),
        pltpu.VMEM((HCH, C), jnp.int32),
        pltpu.VMEM((HCH, C), jnp.int32),
        pltpu.VMEM((C, D), _f32),
        pltpu.VMEM((C, D), _f32),
        pltpu.VMEM((C,), _f32),
        pltpu.VMEM((512,), _f32),
        pltpu.SemaphoreType.DMA,
        pltpu.SemaphoreType.DMA,
    ]
    mesh = plsc.VectorSubcoreMesh(core_axis_name="c", subcore_axis_name="s")

    def body(y, esrc2d, edst2d, *rest):
        if with_count:
            out, cnt_out0, cnt_out1 = rest[0], rest[1], rest[2]
            rest = rest[3:]
        else:
            out = rest[0]
            rest = rest[1:]
        (acc, cntacc, sidx2d, didx2d, rows0, rows1, ones, z1,
         sem0, sem1) = rest
        c = lax.axis_index("c")
        s = lax.axis_index("s")
        w = c * NS + s

        def zrow(r, _):
            for j in range(8):
                rows0[r, pl.ds(j * 16, 16)] = _zeros16()
            return 0
        lax.fori_loop(0, C, zrow, 0)
        for k in range(4):
            pltpu.sync_copy(rows0, acc.at[pl.ds(s * RPT + k * C, C)])
        pltpu.sync_copy(rows0.at[pl.ds(0, 112)],
                        acc.at[pl.ds(s * RPT + 4 * C, 112)])

        @pl.when(s == 0)
        def _():
            pltpu.sync_copy(rows0.at[pl.ds(0, 24)],
                            acc.at[pl.ds(NS * RPT, 24)])

        if with_count:
            @pl.when(s == 0)
            def _():
                def z1row(i, _):
                    z1[pl.ds(i * 16, 16)] = _zeros16()
                    return 0
                lax.fori_loop(0, 32, z1row, 0)
                for k in range(20):
                    pltpu.sync_copy(z1, cntacc.at[pl.ds(k * 512, 512)])

            for j in range(8):
                ones[pl.ds(j * 16, 16)] = jnp.ones((16,), _f32)

        crow = w * CPW
        plsc.subcore_barrier()

        def do_chunk(j, rows, sem):
            pltpu.make_async_copy(y.at[sidx2d.at[j]], rows, sem).wait()
            pltpu.sync_copy(rows, acc.at[didx2d.at[j]], add=True)
            if with_count:
                pltpu.sync_copy(ones, cntacc.at[didx2d.at[j]], add=True)

            @pl.when(j + 2 < HCH)
            def _():
                pltpu.async_copy(y.at[sidx2d.at[j + 2]], rows, sem)

        def pbody(p, _):
            do_chunk(2 * p, rows0, sem0)
            do_chunk(2 * p + 1, rows1, sem1)
            return 0

        for h in range(CPW // HCH):
            pltpu.sync_copy(esrc2d.at[pl.ds(crow + h * HCH, HCH)], sidx2d)
            pltpu.sync_copy(edst2d.at[pl.ds(crow + h * HCH, HCH)], didx2d)
            pltpu.async_copy(y.at[sidx2d.at[0]], rows0, sem0)
            pltpu.async_copy(y.at[sidx2d.at[1]], rows1, sem1)
            lax.fori_loop(0, HCH // 2, pbody, 0)

        plsc.subcore_barrier()

        pltpu.sync_copy(acc.at[pl.ds(s * RPT, RPT)],
                        out.at[c, pl.ds(s * RPT, RPT)])

        @pl.when(s == 0)
        def _():
            pltpu.sync_copy(acc.at[pl.ds(NS * RPT, 16)],
                            out.at[c, pl.ds(NS * RPT, 16)])
        if with_count:
            @pl.when(jnp.logical_and(s == 0, c == 0))
            def _():
                pltpu.sync_copy(cntacc, cnt_out0)

            @pl.when(jnp.logical_and(s == 0, c == 1))
            def _():
                pltpu.sync_copy(cntacc, cnt_out1)

    return pl.kernel(body, out_type=out_type, mesh=mesh,
                     scratch_types=scratch_types,
                     name="sc_segsum_cnt" if with_count else "sc_segsum")


_sc_segsum_cnt = _make_sc_segsum(True)
_sc_segsum = _make_sc_segsum(False)


BLK = 1000
GRID = N // BLK

_full = lambda shape: pl.BlockSpec(shape, lambda i: tuple(0 for _ in shape))
_rows = lambda: pl.BlockSpec((BLK, D), lambda i: (i, 0))


def _tc1_body(x_ref, w1_ref, b1_ref, w2_ref, h_ref, y_ref):
    x = x_ref[...]
    h_ref[...] = jnp.dot(x, w1_ref[...], preferred_element_type=_f32) + b1_ref[...]
    y_ref[...] = jnp.dot(x, w2_ref[...], preferred_element_type=_f32)


_tc1 = pl.pallas_call(
    _tc1_body,
    grid=(GRID,),
    in_specs=[_rows(), _full((D, D)), _full((1, D)), _full((D, D))],
    out_specs=[_rows(), _rows()],
    out_shape=[jax.ShapeDtypeStruct((N, D), _f32)] * 2,
    compiler_params=pltpu.CompilerParams(dimension_semantics=("parallel",)),
)


def _combine(sp, c0, c1, b2):
    s = sp[0] + sp[1]
    cnt = c0 + c1
    return (s + cnt * b2) / jnp.maximum(cnt, 1.0)


def _tc2_body(h1_ref, sp_ref, c0_ref, c1_ref, b2a_ref, w1_ref, b1_ref,
              w2_ref, h_ref, y_ref):
    mean = _combine(sp_ref[...], c0_ref[...], c1_ref[...], b2a_ref[...])
    x2 = jnp.maximum(h1_ref[...] + mean, 0.0)
    h_ref[...] = jnp.dot(x2, w1_ref[...], preferred_element_type=_f32) + b1_ref[...]
    y_ref[...] = jnp.dot(x2, w2_ref[...], preferred_element_type=_f32)


_tc2 = pl.pallas_call(
    _tc2_body,
    grid=(GRID,),
    in_specs=[_rows(),
              pl.BlockSpec((2, BLK, D), lambda i: (0, i, 0)),
              pl.BlockSpec((BLK, 1), lambda i: (i, 0)),
              pl.BlockSpec((BLK, 1), lambda i: (i, 0)),
              _full((1, D)), _full((D, D)), _full((1, D)), _full((D, D))],
    out_specs=[_rows(), _rows()],
    out_shape=[jax.ShapeDtypeStruct((N, D), _f32)] * 2,
    compiler_params=pltpu.CompilerParams(dimension_semantics=("parallel",)),
)


def _tc3_body(h2_ref, sp_ref, c0_ref, c1_ref, b2b_ref, o_ref):
    mean = _combine(sp_ref[...], c0_ref[...], c1_ref[...], b2b_ref[...])
    o_ref[...] = h2_ref[...] + mean


_tc3 = pl.pallas_call(
    _tc3_body,
    grid=(GRID,),
    in_specs=[_rows(),
              pl.BlockSpec((2, BLK, D), lambda i: (0, i, 0)),
              pl.BlockSpec((BLK, 1), lambda i: (i, 0)),
              pl.BlockSpec((BLK, 1), lambda i: (i, 0)),
              _full((1, D))],
    out_specs=_rows(),
    out_shape=jax.ShapeDtypeStruct((N, D), _f32),
    compiler_params=pltpu.CompilerParams(dimension_semantics=("parallel",)),
)


def kernel(inp, edge_index, W1a, b1a, W2a, b2a, W1b, b1b, W2b, b2b):
    ei = edge_index.astype(jnp.int32)
    esrc2d = jnp.concatenate(
        [ei[0], jnp.zeros((EPAD,), jnp.int32)]).reshape(NCHUNKP, C)
    edst2d = jnp.concatenate(
        [ei[1], jnp.full((EPAD,), N, jnp.int32)]).reshape(NCHUNKP, C)
    h1, y1 = _tc1(inp, W1a, b1a.reshape(1, D), W2a)
    sp1, cnt0, cnt1 = _sc_segsum_cnt(y1, esrc2d, edst2d)
    c0 = cnt0[:N].reshape(N, 1)
    c1 = cnt1[:N].reshape(N, 1)
    h2, y2 = _tc2(h1, sp1, c0, c1, b2a.reshape(1, D), W1b,
                  b1b.reshape(1, D), W2b)
    (sp2,) = _sc_segsum(y2, esrc2d, edst2d)
    return _tc3(h2, sp2, c0, c1, b2b.reshape(1, D))


# speedup vs baseline: 1.0047x; 1.0047x over previous
import functools

import jax
import jax.numpy as jnp
from jax import lax
from jax.experimental import pallas as pl
from jax.experimental.pallas import tpu as pltpu
from jax.experimental.pallas import tpu_sc as plsc

N = 10000
E = 320000
D = 128

NC = 2
NS = 16
NW = NC * NS
C = 128
CPW = 80
NCHUNKP = NW * CPW
EPAD = NCHUNKP * C - E
NACC = N + C
NCNT = 10240
RPT = 624
HCH = CPW // 2

_f32 = jnp.float32


def _zeros16():
    return jnp.zeros((16,), _f32)


def _make_sc_segsum(with_count):
    out_type = [jax.ShapeDtypeStruct((NC, N, D), _f32)]
    if with_count:
        out_type.append(jax.ShapeDtypeStruct((NCNT,), _f32))
        out_type.append(jax.ShapeDtypeStruct((NCNT,), _f32))

    scratch_types = [
        pltpu.VMEM_SHARED((NACC, D), _f32),
        pltpu.VMEM_SHARED((NCNT,), _f32),
        pltpu.VMEM((HCH, C), jnp.int32),
        pltpu.VMEM((HCH, C), jnp.int32),
        pltpu.VMEM((C, D), _f32),
        pltpu.VMEM((C, D), _f32),
        pltpu.VMEM((C,), _f32),
        pltpu.VMEM((512,), _f32),
        pltpu.SemaphoreType.DMA,
        pltpu.SemaphoreType.DMA,
    ]
    mesh = plsc.VectorSubcoreMesh(core_axis_name="c", subcore_axis_name="s")

    def body(y, esrc2d, edst2d, *rest):
        if with_count:
            out, cnt_out0, cnt_out1 = rest[0], rest[1], rest[2]
            rest = rest[3:]
        else:
            out = rest[0]
            rest = rest[1:]
        (acc, cntacc, sidx2d, didx2d, rows0, rows1, ones, z1,
         sem0, sem1) = rest
        c = lax.axis_index("c")
        s = lax.axis_index("s")
        w = c * NS + s

        def zrow(r, _):
            for j in range(8):
                rows0[r, pl.ds(j * 16, 16)] = _zeros16()
            return 0
        lax.fori_loop(0, C, zrow, 0)
        for k in range(4):
            pltpu.sync_copy(rows0, acc.at[pl.ds(s * RPT + k * C, C)])
        pltpu.sync_copy(rows0.at[pl.ds(0, 112)],
                        acc.at[pl.ds(s * RPT + 4 * C, 112)])

        @pl.when(s == 0)
        def _():
            pltpu.sync_copy(rows0.at[pl.ds(0, 24)],
                            acc.at[pl.ds(NS * RPT, 24)])

        if with_count:
            @pl.when(s == 0)
            def _():
                def z1row(i, _):
                    z1[pl.ds(i * 16, 16)] = _zeros16()
                    return 0
                lax.fori_loop(0, 32, z1row, 0)
                for k in range(20):
                    pltpu.sync_copy(z1, cntacc.at[pl.ds(k * 512, 512)])

            for j in range(8):
                ones[pl.ds(j * 16, 16)] = jnp.ones((16,), _f32)

        crow = w * CPW
        plsc.subcore_barrier()

        def do_chunk(j, rows, sem):
            pltpu.make_async_copy(y.at[sidx2d.at[j]], rows, sem).wait()
            pltpu.sync_copy(rows, acc.at[didx2d.at[j]], add=True)
            if with_count:
                pltpu.sync_copy(ones, cntacc.at[didx2d.at[j]], add=True)

            @pl.when(j + 2 < HCH)
            def _():
                pltpu.async_copy(y.at[sidx2d.at[j + 2]], rows, sem)

        def pbody(p, _):
            do_chunk(2 * p, rows0, sem0)
            do_chunk(2 * p + 1, rows1, sem1)
            return 0

        for h in range(CPW // HCH):
            pltpu.sync_copy(esrc2d.at[pl.ds(crow + h * HCH, HCH)], sidx2d)
            pltpu.sync_copy(edst2d.at[pl.ds(crow + h * HCH, HCH)], didx2d)
            pltpu.async_copy(y.at[sidx2d.at[0]], rows0, sem0)
            pltpu.async_copy(y.at[sidx2d.at[1]], rows1, sem1)
            lax.fori_loop(0, HCH // 2, pbody, 0)

        plsc.subcore_barrier()

        pltpu.sync_copy(acc.at[pl.ds(s * RPT, RPT)],
                        out.at[c, pl.ds(s * RPT, RPT)])

        @pl.when(s == 0)
        def _():
            pltpu.sync_copy(acc.at[pl.ds(NS * RPT, 16)],
                            out.at[c, pl.ds(NS * RPT, 16)])
        if with_count:
            @pl.when(jnp.logical_and(s == 0, c == 0))
            def _():
                pltpu.sync_copy(cntacc, cnt_out0)

            @pl.when(jnp.logical_and(s == 0, c == 1))
            def _():
                pltpu.sync_copy(cntacc, cnt_out1)

    return pl.kernel(body, out_type=out_type, mesh=mesh,
                     scratch_types=scratch_types,
                     name="sc_segsum_cnt" if with_count else "sc_segsum")


_sc_segsum_cnt = _make_sc_segsum(True)
_sc_segsum = _make_sc_segsum(False)


BLK = 1000
GRID = N // BLK

_full = lambda shape: pl.BlockSpec(shape, lambda i: tuple(0 for _ in shape))
_rows = lambda: pl.BlockSpec((BLK, D), lambda i: (i, 0))


def _tc1_body(x_ref, w1_ref, b1_ref, w2_ref, h_ref, y_ref):
    x = x_ref[...]
    h_ref[...] = jnp.dot(x, w1_ref[...], preferred_element_type=_f32) + b1_ref[...]
    y_ref[...] = jnp.dot(x, w2_ref[...], preferred_element_type=_f32)


_tc1 = pl.pallas_call(
    _tc1_body,
    grid=(GRID,),
    in_specs=[_rows(), _full((D, D)), _full((1, D)), _full((D, D))],
    out_specs=[_rows(), _rows()],
    out_shape=[jax.ShapeDtypeStruct((N, D), _f32)] * 2,
    compiler_params=pltpu.CompilerParams(dimension_semantics=("parallel",)),
)


def _combine(sp, c0, c1, b2):
    s = sp[0] + sp[1]
    cnt = c0 + c1
    return (s + cnt * b2) / jnp.maximum(cnt, 1.0)


def _tc2_body(h1_ref, sp_ref, c0_ref, c1_ref, b2a_ref, w1_ref, b1_ref,
              w2_ref, h_ref, y_ref):
    mean = _combine(sp_ref[...], c0_ref[...], c1_ref[...], b2a_ref[...])
    x2 = jnp.maximum(h1_ref[...] + mean, 0.0)
    h_ref[...] = jnp.dot(x2, w1_ref[...], preferred_element_type=_f32) + b1_ref[...]
    y_ref[...] = jnp.dot(x2, w2_ref[...], preferred_element_type=_f32)


_tc2 = pl.pallas_call(
    _tc2_body,
    grid=(GRID,),
    in_specs=[_rows(),
              pl.BlockSpec((2, BLK, D), lambda i: (0, i, 0)),
              pl.BlockSpec((BLK, 1), lambda i: (i, 0)),
              pl.BlockSpec((BLK, 1), lambda i: (i, 0)),
              _full((1, D)), _full((D, D)), _full((1, D)), _full((D, D))],
    out_specs=[_rows(), _rows()],
    out_shape=[jax.ShapeDtypeStruct((N, D), _f32)] * 2,
    compiler_params=pltpu.CompilerParams(dimension_semantics=("parallel",)),
)


def _tc3_body(h2_ref, sp_ref, c0_ref, c1_ref, b2b_ref, o_ref):
    mean = _combine(sp_ref[...], c0_ref[...], c1_ref[...], b2b_ref[...])
    o_ref[...] = h2_ref[...] + mean


_tc3 = pl.pallas_call(
    _tc3_body,
    grid=(GRID,),
    in_specs=[_rows(),
              pl.BlockSpec((2, BLK, D), lambda i: (0, i, 0)),
              pl.BlockSpec((BLK, 1), lambda i: (i, 0)),
              pl.BlockSpec((BLK, 1), lambda i: (i, 0)),
              _full((1, D))],
    out_specs=_rows(),
    out_shape=jax.ShapeDtypeStruct((N, D), _f32),
    compiler_params=pltpu.CompilerParams(dimension_semantics=("parallel",)),
)


def kernel(inp, edge_index, W1a, b1a, W2a, b2a, W1b, b1b, W2b, b2b):
    ei = edge_index.astype(jnp.int32)
    esrc2d = jnp.concatenate(
        [ei[0], jnp.zeros((EPAD,), jnp.int32)]).reshape(NCHUNKP, C)
    edst2d = jnp.concatenate(
        [ei[1], N + (jnp.arange(EPAD, dtype=jnp.int32) % C)]
    ).reshape(NCHUNKP, C)
    h1, y1 = _tc1(inp, W1a, b1a.reshape(1, D), W2a)
    sp1, cnt0, cnt1 = _sc_segsum_cnt(y1, esrc2d, edst2d)
    c0 = cnt0[:N].reshape(N, 1)
    c1 = cnt1[:N].reshape(N, 1)
    h2, y2 = _tc2(h1, sp1, c0, c1, b2a.reshape(1, D), W1b,
                  b1b.reshape(1, D), W2b)
    (sp2,) = _sc_segsum(y2, esrc2d, edst2d)
    return _tc3(h2, sp2, c0, c1, b2b.reshape(1, D))


# speedup vs baseline: 3.2048x; 3.1897x over previous
import functools

import jax
import jax.numpy as jnp
from jax import lax
from jax.experimental import pallas as pl
from jax.experimental.pallas import tpu as pltpu
from jax.experimental.pallas import tpu_sc as plsc

N = 10000
E = 320000
D = 128

NC = 2
NS = 16
NW = NC * NS
C = 128
CPW = 80
NCHUNKP = NW * CPW
EPAD = NCHUNKP * C - E
NACC = N + C
NCNT = 10240
RPT = 624
HCH = CPW // 2

_f32 = jnp.float32


def _zeros16():
    return jnp.zeros((16,), _f32)


def _make_sc_segsum(with_count):
    out_type = [jax.ShapeDtypeStruct((NC, N, D), _f32)]
    if with_count:
        out_type.append(jax.ShapeDtypeStruct((NCNT,), _f32))
        out_type.append(jax.ShapeDtypeStruct((NCNT,), _f32))

    scratch_types = [
        pltpu.VMEM_SHARED((NACC, D), _f32),
        pltpu.VMEM_SHARED((NCNT,), _f32),
        pltpu.VMEM((HCH, C), jnp.int32),
        pltpu.VMEM((HCH, C), jnp.int32),
        pltpu.VMEM((C, D), _f32),
        pltpu.VMEM((C, D), _f32),
        pltpu.VMEM((C,), _f32),
        pltpu.VMEM((512,), _f32),
        pltpu.SemaphoreType.DMA,
        pltpu.SemaphoreType.DMA,
    ]
    mesh = plsc.VectorSubcoreMesh(core_axis_name="c", subcore_axis_name="s")

    def body(y, esrc2d, edst2d, *rest):
        if with_count:
            out, cnt_out0, cnt_out1 = rest[0], rest[1], rest[2]
            rest = rest[3:]
        else:
            out = rest[0]
            rest = rest[1:]
        (acc, cntacc, sidx2d, didx2d, rows0, rows1, ones, z1,
         sem0, sem1) = rest
        c = lax.axis_index("c")
        s = lax.axis_index("s")
        w = c * NS + s

        def zrow(r, _):
            for j in range(8):
                rows0[r, pl.ds(j * 16, 16)] = _zeros16()
            return 0
        lax.fori_loop(0, C, zrow, 0)
        for k in range(4):
            pltpu.sync_copy(rows0, acc.at[pl.ds(s * RPT + k * C, C)])
        pltpu.sync_copy(rows0.at[pl.ds(0, 112)],
                        acc.at[pl.ds(s * RPT + 4 * C, 112)])

        @pl.when(s == 0)
        def _():
            pltpu.sync_copy(rows0.at[pl.ds(0, 24)],
                            acc.at[pl.ds(NS * RPT, 24)])

        if with_count:
            @pl.when(s == 0)
            def _():
                def z1row(i, _):
                    z1[pl.ds(i * 16, 16)] = _zeros16()
                    return 0
                lax.fori_loop(0, 32, z1row, 0)
                for k in range(20):
                    pltpu.sync_copy(z1, cntacc.at[pl.ds(k * 512, 512)])

            for j in range(8):
                ones[pl.ds(j * 16, 16)] = jnp.ones((16,), _f32)

        crow = w * CPW
        plsc.subcore_barrier()

        def do_chunk(j, rows, sem):
            pltpu.make_async_copy(y.at[sidx2d.at[j]], rows, sem).wait()
            pltpu.sync_copy(rows, acc.at[didx2d.at[j]], add=True)
            if with_count:
                pltpu.sync_copy(ones, cntacc.at[didx2d.at[j]], add=True)

            @pl.when(j + 2 < HCH)
            def _():
                pltpu.async_copy(y.at[sidx2d.at[j + 2]], rows, sem)

        def pbody(p, _):
            do_chunk(2 * p, rows0, sem0)
            do_chunk(2 * p + 1, rows1, sem1)
            return 0

        for h in range(CPW // HCH):
            pltpu.sync_copy(esrc2d.at[pl.ds(crow + h * HCH, HCH)], sidx2d)
            pltpu.sync_copy(edst2d.at[pl.ds(crow + h * HCH, HCH)], didx2d)
            pltpu.async_copy(y.at[sidx2d.at[0]], rows0, sem0)
            pltpu.async_copy(y.at[sidx2d.at[1]], rows1, sem1)
            lax.fori_loop(0, HCH // 2, pbody, 0)

        plsc.subcore_barrier()

        pltpu.sync_copy(acc.at[pl.ds(s * RPT, RPT)],
                        out.at[c, pl.ds(s * RPT, RPT)])

        @pl.when(s == 0)
        def _():
            pltpu.sync_copy(acc.at[pl.ds(NS * RPT, 16)],
                            out.at[c, pl.ds(NS * RPT, 16)])
        if with_count:
            @pl.when(jnp.logical_and(s == 0, c == 0))
            def _():
                pltpu.sync_copy(cntacc, cnt_out0)

            @pl.when(jnp.logical_and(s == 0, c == 1))
            def _():
                pltpu.sync_copy(cntacc, cnt_out1)

    return pl.kernel(body, out_type=out_type, mesh=mesh,
                     scratch_types=scratch_types,
                     name="sc_segsum_cnt" if with_count else "sc_segsum")


_sc_segsum_cnt = _make_sc_segsum(True)
_sc_segsum = _make_sc_segsum(False)


BLK = 1000
GRID = N // BLK

_full = lambda shape: pl.BlockSpec(shape, lambda i: tuple(0 for _ in shape))
_rows = lambda: pl.BlockSpec((BLK, D), lambda i: (i, 0))


def _tc1_body(x_ref, w1_ref, b1_ref, w2_ref, h_ref, y_ref):
    x = x_ref[...]
    h_ref[...] = jnp.dot(x, w1_ref[...], preferred_element_type=_f32) + b1_ref[...]
    y_ref[...] = jnp.dot(x, w2_ref[...], preferred_element_type=_f32)


_tc1 = pl.pallas_call(
    _tc1_body,
    grid=(GRID,),
    in_specs=[_rows(), _full((D, D)), _full((1, D)), _full((D, D))],
    out_specs=[_rows(), _rows()],
    out_shape=[jax.ShapeDtypeStruct((N, D), _f32)] * 2,
    compiler_params=pltpu.CompilerParams(dimension_semantics=("parallel",)),
)


def _combine(sp, c0, c1, b2):
    s = sp[0] + sp[1]
    cnt = c0 + c1
    return (s + cnt * b2) / jnp.maximum(cnt, 1.0)


def _tc2_body(h1_ref, sp_ref, c0_ref, c1_ref, b2a_ref, w1_ref, b1_ref,
              w2_ref, h_ref, y_ref):
    mean = _combine(sp_ref[...], c0_ref[...], c1_ref[...], b2a_ref[...])
    x2 = jnp.maximum(h1_ref[...] + mean, 0.0)
    h_ref[...] = jnp.dot(x2, w1_ref[...], preferred_element_type=_f32) + b1_ref[...]
    y_ref[...] = jnp.dot(x2, w2_ref[...], preferred_element_type=_f32)


_tc2 = pl.pallas_call(
    _tc2_body,
    grid=(GRID,),
    in_specs=[_rows(),
              pl.BlockSpec((2, BLK, D), lambda i: (0, i, 0)),
              pl.BlockSpec((BLK, 1), lambda i: (i, 0)),
              pl.BlockSpec((BLK, 1), lambda i: (i, 0)),
              _full((1, D)), _full((D, D)), _full((1, D)), _full((D, D))],
    out_specs=[_rows(), _rows()],
    out_shape=[jax.ShapeDtypeStruct((N, D), _f32)] * 2,
    compiler_params=pltpu.CompilerParams(dimension_semantics=("parallel",)),
)


def _tc3_body(h2_ref, sp_ref, c0_ref, c1_ref, b2b_ref, o_ref):
    mean = _combine(sp_ref[...], c0_ref[...], c1_ref[...], b2b_ref[...])
    o_ref[...] = h2_ref[...] + mean


_tc3 = pl.pallas_call(
    _tc3_body,
    grid=(GRID,),
    in_specs=[_rows(),
              pl.BlockSpec((2, BLK, D), lambda i: (0, i, 0)),
              pl.BlockSpec((BLK, 1), lambda i: (i, 0)),
              pl.BlockSpec((BLK, 1), lambda i: (i, 0)),
              _full((1, D))],
    out_specs=_rows(),
    out_shape=jax.ShapeDtypeStruct((N, D), _f32),
    compiler_params=pltpu.CompilerParams(dimension_semantics=("parallel",)),
)


def kernel(inp, edge_index, W1a, b1a, W2a, b2a, W1b, b1b, W2b, b2b):
    ei = edge_index.astype(jnp.int32)
    esrc2d = jnp.concatenate(
        [ei[0], jnp.arange(EPAD, dtype=jnp.int32) % C]
    ).reshape(NCHUNKP, C)
    edst2d = jnp.concatenate(
        [ei[1], N + (jnp.arange(EPAD, dtype=jnp.int32) % C)]
    ).reshape(NCHUNKP, C)
    h1, y1 = _tc1(inp, W1a, b1a.reshape(1, D), W2a)
    sp1, cnt0, cnt1 = _sc_segsum_cnt(y1, esrc2d, edst2d)
    c0 = cnt0[:N].reshape(N, 1)
    c1 = cnt1[:N].reshape(N, 1)
    h2, y2 = _tc2(h1, sp1, c0, c1, b2a.reshape(1, D), W1b,
                  b1b.reshape(1, D), W2b)
    (sp2,) = _sc_segsum(y2, esrc2d, edst2d)
    return _tc3(h2, sp2, c0, c1, b2b.reshape(1, D))


# speedup vs baseline: 3.2058x; 1.0003x over previous
import functools

import jax
import jax.numpy as jnp
from jax import lax
from jax.experimental import pallas as pl
from jax.experimental.pallas import tpu as pltpu
from jax.experimental.pallas import tpu_sc as plsc

N = 10000
E = 320000
D = 128

NC = 2
NS = 16
NW = NC * NS
C = 128
CPW = 80
NCHUNKP = NW * CPW
EPAD = NCHUNKP * C - E
NACC = N + C
NCNT = 10240
RPT = 624
HCH = CPW // 2

_f32 = jnp.float32
_bf16 = jnp.bfloat16


def _zeros16():
    return jnp.zeros((16,), _f32)


def _make_sc_segsum(with_count):
    out_type = [jax.ShapeDtypeStruct((NC, N, D), _f32)]
    if with_count:
        out_type.append(jax.ShapeDtypeStruct((NCNT,), _f32))
        out_type.append(jax.ShapeDtypeStruct((NCNT,), _f32))

    scratch_types = [
        pltpu.VMEM_SHARED((NACC, D), _f32),
        pltpu.VMEM_SHARED((NCNT,), _f32),
        pltpu.VMEM((HCH, C), jnp.int32),
        pltpu.VMEM((HCH, C), jnp.int32),
        pltpu.VMEM((C, D), _f32),
        pltpu.VMEM((C, D), _f32),
        pltpu.VMEM((C,), _f32),
        pltpu.VMEM((512,), _f32),
        pltpu.SemaphoreType.DMA,
        pltpu.SemaphoreType.DMA,
    ]
    mesh = plsc.VectorSubcoreMesh(core_axis_name="c", subcore_axis_name="s")

    def body(y, esrc2d, edst2d, *rest):
        if with_count:
            out, cnt_out0, cnt_out1 = rest[0], rest[1], rest[2]
            rest = rest[3:]
        else:
            out = rest[0]
            rest = rest[1:]
        (acc, cntacc, sidx2d, didx2d, rows0, rows1, ones, z1,
         sem0, sem1) = rest
        c = lax.axis_index("c")
        s = lax.axis_index("s")
        w = c * NS + s

        def zrow(r, _):
            for j in range(8):
                rows0[r, pl.ds(j * 16, 16)] = _zeros16()
            return 0
        lax.fori_loop(0, C, zrow, 0)
        for k in range(4):
            pltpu.sync_copy(rows0, acc.at[pl.ds(s * RPT + k * C, C)])
        pltpu.sync_copy(rows0.at[pl.ds(0, 112)],
                        acc.at[pl.ds(s * RPT + 4 * C, 112)])

        @pl.when(s == 0)
        def _():
            pltpu.sync_copy(rows0.at[pl.ds(0, 24)],
                            acc.at[pl.ds(NS * RPT, 24)])

        if with_count:
            @pl.when(s == 0)
            def _():
                def z1row(i, _):
                    z1[pl.ds(i * 16, 16)] = _zeros16()
                    return 0
                lax.fori_loop(0, 32, z1row, 0)
                for k in range(20):
                    pltpu.sync_copy(z1, cntacc.at[pl.ds(k * 512, 512)])

            for j in range(8):
                ones[pl.ds(j * 16, 16)] = jnp.ones((16,), _f32)

        crow = w * CPW
        plsc.subcore_barrier()

        def do_chunk(j, rows, sem):
            pltpu.make_async_copy(y.at[sidx2d.at[j]], rows, sem).wait()
            pltpu.sync_copy(rows, acc.at[didx2d.at[j]], add=True)
            if with_count:
                pltpu.sync_copy(ones, cntacc.at[didx2d.at[j]], add=True)

            @pl.when(j + 2 < HCH)
            def _():
                pltpu.async_copy(y.at[sidx2d.at[j + 2]], rows, sem)

        def pbody(p, _):
            do_chunk(2 * p, rows0, sem0)
            do_chunk(2 * p + 1, rows1, sem1)
            return 0

        for h in range(CPW // HCH):
            pltpu.sync_copy(esrc2d.at[pl.ds(crow + h * HCH, HCH)], sidx2d)
            pltpu.sync_copy(edst2d.at[pl.ds(crow + h * HCH, HCH)], didx2d)
            pltpu.async_copy(y.at[sidx2d.at[0]], rows0, sem0)
            pltpu.async_copy(y.at[sidx2d.at[1]], rows1, sem1)
            lax.fori_loop(0, HCH // 2, pbody, 0)

        plsc.subcore_barrier()

        pltpu.sync_copy(acc.at[pl.ds(s * RPT, RPT)],
                        out.at[c, pl.ds(s * RPT, RPT)])

        @pl.when(s == 0)
        def _():
            pltpu.sync_copy(acc.at[pl.ds(NS * RPT, 16)],
                            out.at[c, pl.ds(NS * RPT, 16)])
        if with_count:
            @pl.when(jnp.logical_and(s == 0, c == 0))
            def _():
                pltpu.sync_copy(cntacc, cnt_out0)

            @pl.when(jnp.logical_and(s == 0, c == 1))
            def _():
                pltpu.sync_copy(cntacc, cnt_out1)

    return pl.kernel(body, out_type=out_type, mesh=mesh,
                     scratch_types=scratch_types,
                     name="sc_segsum_cnt" if with_count else "sc_segsum")


_sc_segsum_cnt = _make_sc_segsum(True)
_sc_segsum = _make_sc_segsum(False)


BLK = 1000
GRID = N // BLK

_full = lambda shape: pl.BlockSpec(shape, lambda i: tuple(0 for _ in shape))
_rows = lambda: pl.BlockSpec((BLK, D), lambda i: (i, 0))


def _tc1_body(x_ref, w1_ref, b1_ref, w2_ref, h_ref, y_ref):
    x = x_ref[...]
    h_ref[...] = jnp.dot(x, w1_ref[...], preferred_element_type=_f32) + b1_ref[...]
    y_ref[...] = jnp.dot(x, w2_ref[...], preferred_element_type=_f32)


_tc1 = pl.pallas_call(
    _tc1_body,
    grid=(GRID,),
    in_specs=[_rows(), _full((D, D)), _full((1, D)), _full((D, D))],
    out_specs=[_rows(), _rows()],
    out_shape=[jax.ShapeDtypeStruct((N, D), _f32)] * 2,
    compiler_params=pltpu.CompilerParams(dimension_semantics=("parallel",)),
)


def _combine(sp, c0, c1, b2):
    s = sp[0] + sp[1]
    cnt = c0 + c1
    return (s + cnt * b2) / jnp.maximum(cnt, 1.0)


def _tc2_body(h1_ref, sp_ref, c0_ref, c1_ref, b2a_ref, w1_ref, b1_ref,
              w2_ref, h_ref, y_ref):
    mean = _combine(sp_ref[...], c0_ref[...], c1_ref[...], b2a_ref[...])
    x2 = jnp.maximum(h1_ref[...] + mean, 0.0)
    h_ref[...] = jnp.dot(x2, w1_ref[...], preferred_element_type=_f32) + b1_ref[...]
    y_ref[...] = jnp.dot(x2, w2_ref[...], preferred_element_type=_f32)


_tc2 = pl.pallas_call(
    _tc2_body,
    grid=(GRID,),
    in_specs=[_rows(),
              pl.BlockSpec((2, BLK, D), lambda i: (0, i, 0)),
              pl.BlockSpec((BLK, 1), lambda i: (i, 0)),
              pl.BlockSpec((BLK, 1), lambda i: (i, 0)),
              _full((1, D)), _full((D, D)), _full((1, D)), _full((D, D))],
    out_specs=[_rows(), _rows()],
    out_shape=[jax.ShapeDtypeStruct((N, D), _f32)] * 2,
    compiler_params=pltpu.CompilerParams(dimension_semantics=("parallel",)),
)


def _tc3_body(h2_ref, sp_ref, c0_ref, c1_ref, b2b_ref, o_ref):
    mean = _combine(sp_ref[...], c0_ref[...], c1_ref[...], b2b_ref[...])
    o_ref[...] = h2_ref[...] + mean


_tc3 = pl.pallas_call(
    _tc3_body,
    grid=(GRID,),
    in_specs=[_rows(),
              pl.BlockSpec((2, BLK, D), lambda i: (0, i, 0)),
              pl.BlockSpec((BLK, 1), lambda i: (i, 0)),
              pl.BlockSpec((BLK, 1), lambda i: (i, 0)),
              _full((1, D))],
    out_specs=_rows(),
    out_shape=jax.ShapeDtypeStruct((N, D), _f32),
    compiler_params=pltpu.CompilerParams(dimension_semantics=("parallel",)),
)


def kernel(inp, edge_index, W1a, b1a, W2a, b2a, W1b, b1b, W2b, b2b):
    ei = edge_index.astype(jnp.int32)
    esrc2d = jnp.concatenate(
        [ei[0], jnp.arange(EPAD, dtype=jnp.int32) % C]
    ).reshape(NCHUNKP, C)
    edst2d = jnp.concatenate(
        [ei[1], N + (jnp.arange(EPAD, dtype=jnp.int32) % C)]
    ).reshape(NCHUNKP, C)
    h1, y1 = _tc1(inp, W1a, b1a.reshape(1, D), W2a)
    sp1, cnt0, cnt1 = _sc_segsum_cnt(y1, esrc2d, edst2d)
    c0 = cnt0[:N].reshape(N, 1)
    c1 = cnt1[:N].reshape(N, 1)
    h2, y2 = _tc2(h1, sp1, c0, c1, b2a.reshape(1, D), W1b,
                  b1b.reshape(1, D), W2b)
    (sp2,) = _sc_segsum(y2, esrc2d, edst2d)
    return _tc3(h2, sp2, c0, c1, b2b.reshape(1, D))


# speedup vs baseline: 3.3406x; 1.0420x over previous
import functools

import jax
import jax.numpy as jnp
from jax import lax
from jax.experimental import pallas as pl
from jax.experimental.pallas import tpu as pltpu
from jax.experimental.pallas import tpu_sc as plsc

N = 10000
E = 320000
D = 128

NC = 2
NS = 16
NW = NC * NS
C = 128
CPW = 80
NCHUNKP = NW * CPW
EPAD = NCHUNKP * C - E
NACC = N + C
NCNT = 10240
RPT = 624
HCH = CPW // 2

_f32 = jnp.float32
_bf16 = jnp.bfloat16


def _zeros16():
    return jnp.zeros((16,), _f32)


def _make_sc_segsum(with_count):
    out_type = [jax.ShapeDtypeStruct((NC, N, D), _f32)]
    if with_count:
        out_type.append(jax.ShapeDtypeStruct((NCNT,), _f32))
        out_type.append(jax.ShapeDtypeStruct((NCNT,), _f32))

    scratch_types = [
        pltpu.VMEM_SHARED((NACC, D), _f32),
        pltpu.VMEM_SHARED((NCNT,), _f32),
        pltpu.VMEM((HCH, C), jnp.int32),
        pltpu.VMEM((HCH, C), jnp.int32),
        pltpu.VMEM((C, D), _f32),
        pltpu.VMEM((C, D), _f32),
        pltpu.VMEM((C,), _f32),
        pltpu.VMEM((512,), _f32),
        pltpu.SemaphoreType.DMA,
        pltpu.SemaphoreType.DMA,
    ]
    mesh = plsc.VectorSubcoreMesh(core_axis_name="c", subcore_axis_name="s")

    def body(y, esrc2d, edst2d, *rest):
        if with_count:
            out, cnt_out0, cnt_out1 = rest[0], rest[1], rest[2]
            rest = rest[3:]
        else:
            out = rest[0]
            rest = rest[1:]
        (acc, cntacc, sidx2d, didx2d, rows0, rows1, ones, z1,
         sem0, sem1) = rest
        c = lax.axis_index("c")
        s = lax.axis_index("s")
        w = c * NS + s

        def zrow(r, _):
            for j in range(8):
                rows0[r, pl.ds(j * 16, 16)] = _zeros16()
            return 0
        lax.fori_loop(0, C, zrow, 0)
        for k in range(4):
            pltpu.sync_copy(rows0, acc.at[pl.ds(s * RPT + k * C, C)])
        pltpu.sync_copy(rows0.at[pl.ds(0, 112)],
                        acc.at[pl.ds(s * RPT + 4 * C, 112)])

        @pl.when(s == 0)
        def _():
            pltpu.sync_copy(rows0.at[pl.ds(0, 24)],
                            acc.at[pl.ds(NS * RPT, 24)])

        if with_count:
            @pl.when(s == 0)
            def _():
                def z1row(i, _):
                    z1[pl.ds(i * 16, 16)] = _zeros16()
                    return 0
                lax.fori_loop(0, 32, z1row, 0)
                for k in range(20):
                    pltpu.sync_copy(z1, cntacc.at[pl.ds(k * 512, 512)])

            for j in range(8):
                ones[pl.ds(j * 16, 16)] = jnp.ones((16,), _f32)

        crow = w * CPW
        plsc.subcore_barrier()

        def do_chunk(j, rows, sem):
            pltpu.make_async_copy(y.at[sidx2d.at[j]], rows, sem).wait()
            pltpu.sync_copy(rows, acc.at[didx2d.at[j]], add=True)
            if with_count:
                pltpu.sync_copy(ones, cntacc.at[didx2d.at[j]], add=True)

            @pl.when(j + 2 < HCH)
            def _():
                pltpu.async_copy(y.at[sidx2d.at[j + 2]], rows, sem)

        def pbody(p, _):
            do_chunk(2 * p, rows0, sem0)
            do_chunk(2 * p + 1, rows1, sem1)
            return 0

        for h in range(CPW // HCH):
            pltpu.sync_copy(esrc2d.at[pl.ds(crow + h * HCH, HCH)], sidx2d)
            pltpu.sync_copy(edst2d.at[pl.ds(crow + h * HCH, HCH)], didx2d)
            pltpu.async_copy(y.at[sidx2d.at[0]], rows0, sem0)
            pltpu.async_copy(y.at[sidx2d.at[1]], rows1, sem1)
            lax.fori_loop(0, HCH // 2, pbody, 0)

        plsc.subcore_barrier()

        pltpu.sync_copy(acc.at[pl.ds(s * RPT, RPT)],
                        out.at[c, pl.ds(s * RPT, RPT)])

        @pl.when(s == 0)
        def _():
            pltpu.sync_copy(acc.at[pl.ds(NS * RPT, 16)],
                            out.at[c, pl.ds(NS * RPT, 16)])
        if with_count:
            @pl.when(jnp.logical_and(s == 0, c == 0))
            def _():
                pltpu.sync_copy(cntacc, cnt_out0)

            @pl.when(jnp.logical_and(s == 0, c == 1))
            def _():
                pltpu.sync_copy(cntacc, cnt_out1)

    return pl.kernel(body, out_type=out_type, mesh=mesh,
                     scratch_types=scratch_types,
                     name="sc_segsum_cnt" if with_count else "sc_segsum")


_sc_segsum_cnt = _make_sc_segsum(True)
_sc_segsum = _make_sc_segsum(False)


BLK = 2000
GRID = N // BLK

_full = lambda shape: pl.BlockSpec(shape, lambda i: tuple(0 for _ in shape))
_rows = lambda: pl.BlockSpec((BLK, D), lambda i: (i, 0))


def _tc1_body(x_ref, w1_ref, b1_ref, w2_ref, h_ref, y_ref):
    x = x_ref[...]
    h_ref[...] = jnp.dot(x, w1_ref[...], preferred_element_type=_f32) + b1_ref[...]
    y_ref[...] = jnp.dot(x, w2_ref[...], preferred_element_type=_f32)


_tc1 = pl.pallas_call(
    _tc1_body,
    grid=(GRID,),
    in_specs=[_rows(), _full((D, D)), _full((1, D)), _full((D, D))],
    out_specs=[_rows(), _rows()],
    out_shape=[jax.ShapeDtypeStruct((N, D), _f32)] * 2,
    compiler_params=pltpu.CompilerParams(dimension_semantics=("parallel",)),
)


def _combine(sp, cnt, b2):
    s = sp[0] + sp[1]
    return (s + cnt * b2) / jnp.maximum(cnt, 1.0)


def _tc2_body(h1_ref, sp_ref, cnt_ref, b2a_ref, w1_ref, b1_ref,
              w2_ref, h_ref, y_ref):
    mean = _combine(sp_ref[...], cnt_ref[...], b2a_ref[...])
    x2 = jnp.maximum(h1_ref[...] + mean, 0.0)
    h_ref[...] = jnp.dot(x2, w1_ref[...], preferred_element_type=_f32) + b1_ref[...]
    y_ref[...] = jnp.dot(x2, w2_ref[...], preferred_element_type=_f32)


_tc2 = pl.pallas_call(
    _tc2_body,
    grid=(GRID,),
    in_specs=[_rows(),
              pl.BlockSpec((2, BLK, D), lambda i: (0, i, 0)),
              pl.BlockSpec((BLK, 1), lambda i: (i, 0)),
              _full((1, D)), _full((D, D)), _full((1, D)), _full((D, D))],
    out_specs=[_rows(), _rows()],
    out_shape=[jax.ShapeDtypeStruct((N, D), _f32)] * 2,
    compiler_params=pltpu.CompilerParams(dimension_semantics=("parallel",)),
)


def _tc3_body(h2_ref, sp_ref, cnt_ref, b2b_ref, o_ref):
    mean = _combine(sp_ref[...], cnt_ref[...], b2b_ref[...])
    o_ref[...] = h2_ref[...] + mean


_tc3 = pl.pallas_call(
    _tc3_body,
    grid=(GRID,),
    in_specs=[_rows(),
              pl.BlockSpec((2, BLK, D), lambda i: (0, i, 0)),
              pl.BlockSpec((BLK, 1), lambda i: (i, 0)),
              _full((1, D))],
    out_specs=_rows(),
    out_shape=jax.ShapeDtypeStruct((N, D), _f32),
    compiler_params=pltpu.CompilerParams(dimension_semantics=("parallel",)),
)


def kernel(inp, edge_index, W1a, b1a, W2a, b2a, W1b, b1b, W2b, b2b):
    ei = edge_index.astype(jnp.int32)
    esrc2d = jnp.concatenate(
        [ei[0], jnp.arange(EPAD, dtype=jnp.int32) % C]
    ).reshape(NCHUNKP, C)
    edst2d = jnp.concatenate(
        [ei[1], N + (jnp.arange(EPAD, dtype=jnp.int32) % C)]
    ).reshape(NCHUNKP, C)
    h1, y1 = _tc1(inp, W1a, b1a.reshape(1, D), W2a)
    sp1, cnt0, cnt1 = _sc_segsum_cnt(y1, esrc2d, edst2d)
    cnt = (cnt0 + cnt1)[:N].reshape(N, 1)
    h2, y2 = _tc2(h1, sp1, cnt, b2a.reshape(1, D), W1b,
                  b1b.reshape(1, D), W2b)
    (sp2,) = _sc_segsum(y2, esrc2d, edst2d)
    return _tc3(h2, sp2, cnt, b2b.reshape(1, D))


# speedup vs baseline: 3.3534x; 1.0038x over previous
import functools

import jax
import jax.numpy as jnp
from jax import lax
from jax.experimental import pallas as pl
from jax.experimental.pallas import tpu as pltpu
from jax.experimental.pallas import tpu_sc as plsc

N = 10000
E = 320000
D = 128

NC = 2
NS = 16
NW = NC * NS
C = 128
CPW = 80
NCHUNKP = NW * CPW
EPAD = NCHUNKP * C - E
NACC = N + C
NCNT = 10240
RPT = 624
HCH = CPW // 2

_f32 = jnp.float32
_bf16 = jnp.bfloat16


def _zeros16():
    return jnp.zeros((16,), _f32)


def _make_sc_segsum(with_count):
    out_type = [jax.ShapeDtypeStruct((NC, N, D), _f32)]
    if with_count:
        out_type.append(jax.ShapeDtypeStruct((NCNT,), _f32))
        out_type.append(jax.ShapeDtypeStruct((NCNT,), _f32))

    scratch_types = [
        pltpu.VMEM_SHARED((NACC, D), _f32),
        pltpu.VMEM_SHARED((NCNT,), _f32),
        pltpu.VMEM((HCH, C), jnp.int32),
        pltpu.VMEM((HCH, C), jnp.int32),
        pltpu.VMEM((C, D), _f32),
        pltpu.VMEM((C, D), _f32),
        pltpu.VMEM((C,), _f32),
        pltpu.VMEM((512,), _f32),
        pltpu.SemaphoreType.DMA,
        pltpu.SemaphoreType.DMA,
    ]
    mesh = plsc.VectorSubcoreMesh(core_axis_name="c", subcore_axis_name="s")

    def body(y, esrc2d, edst2d, *rest):
        if with_count:
            out, cnt_out0, cnt_out1 = rest[0], rest[1], rest[2]
            rest = rest[3:]
        else:
            out = rest[0]
            rest = rest[1:]
        (acc, cntacc, sidx2d, didx2d, rows0, rows1, ones, z1,
         sem0, sem1) = rest
        c = lax.axis_index("c")
        s = lax.axis_index("s")
        w = c * NS + s

        def zrow(r, _):
            for j in range(8):
                rows0[r, pl.ds(j * 16, 16)] = _zeros16()
            return 0
        lax.fori_loop(0, C, zrow, 0)
        for k in range(4):
            pltpu.sync_copy(rows0, acc.at[pl.ds(s * RPT + k * C, C)])
        pltpu.sync_copy(rows0.at[pl.ds(0, 112)],
                        acc.at[pl.ds(s * RPT + 4 * C, 112)])

        @pl.when(s == 0)
        def _():
            pltpu.sync_copy(rows0.at[pl.ds(0, 24)],
                            acc.at[pl.ds(NS * RPT, 24)])

        if with_count:
            @pl.when(s == 0)
            def _():
                def z1row(i, _):
                    z1[pl.ds(i * 16, 16)] = _zeros16()
                    return 0
                lax.fori_loop(0, 32, z1row, 0)
                for k in range(20):
                    pltpu.sync_copy(z1, cntacc.at[pl.ds(k * 512, 512)])

            for j in range(8):
                ones[pl.ds(j * 16, 16)] = jnp.ones((16,), _f32)

        crow = w * CPW
        plsc.subcore_barrier()

        def do_chunk(j, rows, sem):
            pltpu.make_async_copy(y.at[sidx2d.at[j]], rows, sem).wait()
            pltpu.sync_copy(rows, acc.at[didx2d.at[j]], add=True)
            if with_count:
                pltpu.sync_copy(ones, cntacc.at[didx2d.at[j]], add=True)

            @pl.when(j + 2 < HCH)
            def _():
                pltpu.async_copy(y.at[sidx2d.at[j + 2]], rows, sem)

        def pbody(p, _):
            do_chunk(2 * p, rows0, sem0)
            do_chunk(2 * p + 1, rows1, sem1)
            return 0

        for h in range(CPW // HCH):
            pltpu.sync_copy(esrc2d.at[pl.ds(crow + h * HCH, HCH)], sidx2d)
            pltpu.sync_copy(edst2d.at[pl.ds(crow + h * HCH, HCH)], didx2d)
            pltpu.async_copy(y.at[sidx2d.at[0]], rows0, sem0)
            pltpu.async_copy(y.at[sidx2d.at[1]], rows1, sem1)
            lax.fori_loop(0, HCH // 2, pbody, 0)

        plsc.subcore_barrier()

        pltpu.sync_copy(acc.at[pl.ds(s * RPT, RPT)],
                        out.at[c, pl.ds(s * RPT, RPT)])

        @pl.when(s == 0)
        def _():
            pltpu.sync_copy(acc.at[pl.ds(NS * RPT, 16)],
                            out.at[c, pl.ds(NS * RPT, 16)])
        if with_count:
            @pl.when(jnp.logical_and(s == 0, c == 0))
            def _():
                pltpu.sync_copy(cntacc, cnt_out0)

            @pl.when(jnp.logical_and(s == 0, c == 1))
            def _():
                pltpu.sync_copy(cntacc, cnt_out1)

    return pl.kernel(body, out_type=out_type, mesh=mesh,
                     scratch_types=scratch_types,
                     name="sc_segsum_cnt" if with_count else "sc_segsum")


_sc_segsum_cnt = _make_sc_segsum(True)
_sc_segsum = _make_sc_segsum(False)


BLK = 2000
GRID = N // BLK

_full = lambda shape: pl.BlockSpec(shape, lambda i: tuple(0 for _ in shape))
_rows = lambda: pl.BlockSpec((BLK, D), lambda i: (i, 0))


def _tcy_body(x_ref, w2_ref, y_ref):
    y_ref[...] = jnp.dot(x_ref[...], w2_ref[...], preferred_element_type=_f32)


_tc_y = pl.pallas_call(
    _tcy_body,
    grid=(GRID,),
    in_specs=[_rows(), _full((D, D))],
    out_specs=_rows(),
    out_shape=jax.ShapeDtypeStruct((N, D), _f32),
    compiler_params=pltpu.CompilerParams(dimension_semantics=("parallel",)),
)


def _tch_body(x_ref, w1_ref, b1_ref, h_ref):
    h_ref[...] = jnp.dot(x_ref[...], w1_ref[...],
                         preferred_element_type=_f32) + b1_ref[...]


_tc_h = pl.pallas_call(
    _tch_body,
    grid=(GRID,),
    in_specs=[_rows(), _full((D, D)), _full((1, D))],
    out_specs=_rows(),
    out_shape=jax.ShapeDtypeStruct((N, D), _f32),
    compiler_params=pltpu.CompilerParams(dimension_semantics=("parallel",)),
)


def _combine(sp, cnt, b2):
    s = sp[0] + sp[1]
    return (s + cnt * b2) / jnp.maximum(cnt, 1.0)


def _tc2y_body(h1_ref, sp_ref, cnt_ref, b2a_ref, w2_ref, x2_ref, y_ref):
    mean = _combine(sp_ref[...], cnt_ref[...], b2a_ref[...])
    x2 = jnp.maximum(h1_ref[...] + mean, 0.0)
    x2_ref[...] = x2
    y_ref[...] = jnp.dot(x2, w2_ref[...], preferred_element_type=_f32)


_tc2y = pl.pallas_call(
    _tc2y_body,
    grid=(GRID,),
    in_specs=[_rows(),
              pl.BlockSpec((2, BLK, D), lambda i: (0, i, 0)),
              pl.BlockSpec((BLK, 1), lambda i: (i, 0)),
              _full((1, D)), _full((D, D))],
    out_specs=[_rows(), _rows()],
    out_shape=[jax.ShapeDtypeStruct((N, D), _f32)] * 2,
    compiler_params=pltpu.CompilerParams(dimension_semantics=("parallel",)),
)


def _tc3_body(h2_ref, sp_ref, cnt_ref, b2b_ref, o_ref):
    mean = _combine(sp_ref[...], cnt_ref[...], b2b_ref[...])
    o_ref[...] = h2_ref[...] + mean


_tc3 = pl.pallas_call(
    _tc3_body,
    grid=(GRID,),
    in_specs=[_rows(),
              pl.BlockSpec((2, BLK, D), lambda i: (0, i, 0)),
              pl.BlockSpec((BLK, 1), lambda i: (i, 0)),
              _full((1, D))],
    out_specs=_rows(),
    out_shape=jax.ShapeDtypeStruct((N, D), _f32),
    compiler_params=pltpu.CompilerParams(dimension_semantics=("parallel",)),
)


def kernel(inp, edge_index, W1a, b1a, W2a, b2a, W1b, b1b, W2b, b2b):
    ei = edge_index.astype(jnp.int32)
    esrc2d = jnp.concatenate(
        [ei[0], jnp.arange(EPAD, dtype=jnp.int32) % C]
    ).reshape(NCHUNKP, C)
    edst2d = jnp.concatenate(
        [ei[1], N + (jnp.arange(EPAD, dtype=jnp.int32) % C)]
    ).reshape(NCHUNKP, C)
    y1 = _tc_y(inp, W2a)
    sp1, cnt0, cnt1 = _sc_segsum_cnt(y1, esrc2d, edst2d)
    h1 = _tc_h(inp, W1a, b1a.reshape(1, D))
    cnt = (cnt0 + cnt1)[:N].reshape(N, 1)
    x2, y2 = _tc2y(h1, sp1, cnt, b2a.reshape(1, D), W2b)
    (sp2,) = _sc_segsum(y2, esrc2d, edst2d)
    h2 = _tc_h(x2, W1b, b1b.reshape(1, D))
    return _tc3(h2, sp2, cnt, b2b.reshape(1, D))
